# Initial kernel scaffold; baseline (speedup 1.0000x reference)
import jax
import jax.numpy as jnp
from jax.experimental import pallas as pl


def kernel(x, adj, W1, b1, W2, b2):
    raise NotImplementedError("write your pallas kernel here")


# speedup vs baseline: 5.4062x; 5.4062x over previous
import functools

import jax
import jax.numpy as jnp
from jax import lax
from jax.experimental import pallas as pl
from jax.experimental.pallas import tpu as pltpu
from jax.experimental.pallas import tpu_sc as plsc

N = 10000
E = 320000
D = 128
NC = 2
NS = 16
NW = NC * NS
CH = 80
CHUNKS = E // (NW * CH)
SUB = 4
SUPERS = (CHUNKS - 1) // SUB
CPAD = SUPERS * SUB + SUB
ROWS_PER_TILE = 10240 // NS
NPAD = NS * ROWS_PER_TILE
BM = 400
GRID = N // BM
L = 16


def _log0(x):
    n = jnp.sqrt(jnp.sum(x * x, axis=-1, keepdims=True))
    n = jnp.maximum(n, 1e-7)
    return lax.atan2(n, jnp.ones_like(n)) / n * x


def _exp0(u):
    n = jnp.sqrt(jnp.sum(u * u, axis=-1, keepdims=True))
    n = jnp.maximum(n, 1e-7)
    t = jnp.clip(n, 0.0, 1.5)
    return jnp.tan(t) / n * u


def _enc_body(x_ref, w_ref, b_ref, o_ref):
    u = _log0(x_ref[...])
    o_ref[...] = (
        jnp.dot(u, w_ref[...], preferred_element_type=jnp.float32) + b_ref[...]
    )


def _combine(h_ref, p_ref, d_ref):
    h = h_ref[...]
    s = p_ref[0] + p_ref[1]
    deg = jnp.maximum(jnp.sum(d_ref[...], axis=0), 1.0)
    o = jax.nn.relu(h + s / deg)
    o = o / (jnp.sqrt(jnp.sum(o * o, axis=-1, keepdims=True)) + 1e-7)
    return _exp0(o)


def _comb_enc_body(h_ref, p_ref, d_ref, w_ref, b_ref, o_ref):
    y = _combine(h_ref, p_ref, d_ref)
    u = _log0(y)
    o_ref[...] = (
        jnp.dot(u, w_ref[...], preferred_element_type=jnp.float32) + b_ref[...]
    )


def _final_body(h_ref, p_ref, d_ref, o_ref):
    o_ref[...] = _combine(h_ref, p_ref, d_ref)


_enc = pl.pallas_call(
    _enc_body,
    grid=(GRID,),
    in_specs=[
        pl.BlockSpec((BM, D), lambda i: (i, 0)),
        pl.BlockSpec((D, D), lambda i: (0, 0)),
        pl.BlockSpec((1, D), lambda i: (0, 0)),
    ],
    out_specs=pl.BlockSpec((BM, D), lambda i: (i, 0)),
    out_shape=jax.ShapeDtypeStruct((N, D), jnp.float32),
    compiler_params=pltpu.CompilerParams(
        dimension_semantics=("parallel",),
    ),
)

_comb_enc = pl.pallas_call(
    _comb_enc_body,
    grid=(GRID,),
    in_specs=[
        pl.BlockSpec((BM, D), lambda i: (i, 0)),
        pl.BlockSpec((NC, BM, D), lambda i: (0, i, 0)),
        pl.BlockSpec((NW, BM, 1), lambda i: (0, i, 0)),
        pl.BlockSpec((D, D), lambda i: (0, 0)),
        pl.BlockSpec((1, D), lambda i: (0, 0)),
    ],
    out_specs=pl.BlockSpec((BM, D), lambda i: (i, 0)),
    out_shape=jax.ShapeDtypeStruct((N, D), jnp.float32),
    compiler_params=pltpu.CompilerParams(
        dimension_semantics=("parallel",),
    ),
)

_final = pl.pallas_call(
    _final_body,
    grid=(GRID,),
    in_specs=[
        pl.BlockSpec((BM, D), lambda i: (i, 0)),
        pl.BlockSpec((NC, BM, D), lambda i: (0, i, 0)),
        pl.BlockSpec((NW, BM, 1), lambda i: (0, i, 0)),
    ],
    out_specs=pl.BlockSpec((BM, D), lambda i: (i, 0)),
    out_shape=jax.ShapeDtypeStruct((N, D), jnp.float32),
    compiler_params=pltpu.CompilerParams(
        dimension_semantics=("parallel",),
    ),
)


def _count_deg(dst_v, slot, i, deg_v):
    for k in range(CH // L):
        idx = dst_v[slot, i, pl.ds(k * L, L)]
        cnt, last = plsc.scan_count(idx)
        plsc.addupdate_scatter(
            deg_v, [idx], cnt.astype(jnp.float32), mask=last)


def _sc_agg_body(h_hbm, src_hbm, dst_hbm, z_hbm, out_hbm, deg_hbm,
                 acc, src_v, dst_v, deg_v, r0, r1, s0, s1):
    cid = lax.axis_index("c")
    sid = lax.axis_index("s")
    wid = cid * NS + sid

    zero16 = jnp.zeros((L,), jnp.float32)

    def zrow(i, _):
        deg_v[pl.ds(i * L, L)] = zero16
        return 0

    lax.fori_loop(0, NPAD // L, zrow, 0)

    pltpu.sync_copy(z_hbm, r0)
    for k in range(ROWS_PER_TILE // CH):
        pltpu.sync_copy(
            r0, acc.at[pl.ds(sid * ROWS_PER_TILE + k * CH, CH)])
    plsc.subcore_barrier()

    pltpu.sync_copy(src_hbm.at[wid, pl.ds(0, SUB)], src_v.at[0])
    pltpu.sync_copy(dst_hbm.at[wid, pl.ds(0, SUB)], dst_v.at[0])
    pltpu.async_copy(h_hbm.at[src_v.at[0, 0]], r0, s0)

    def body(s, _):
        cur = lax.rem(s, 2)
        nxt = lax.rem(s + 1, 2)
        pltpu.sync_copy(src_hbm.at[wid, pl.ds((s + 1) * SUB, SUB)],
                        src_v.at[nxt])
        pltpu.sync_copy(dst_hbm.at[wid, pl.ds((s + 1) * SUB, SUB)],
                        dst_v.at[nxt])
        for i in range(SUB):
            rb, sb = (r0, s0) if i % 2 == 0 else (r1, s1)
            nrb, nsb = (r1, s1) if i % 2 == 0 else (r0, s0)
            if i < SUB - 1:
                pltpu.async_copy(h_hbm.at[src_v.at[cur, i + 1]], nrb, nsb)
            else:
                pltpu.async_copy(h_hbm.at[src_v.at[nxt, 0]], nrb, nsb)
            _count_deg(dst_v, cur, i, deg_v)
            pltpu.make_async_copy(h_hbm.at[src_v.at[cur, i]], rb, sb).wait()
            pltpu.sync_copy(rb, acc.at[dst_v.at[cur, i]], add=True)
        return 0

    lax.fori_loop(0, SUPERS, body, 0)
    cur = SUPERS % 2
    _count_deg(dst_v, cur, 0, deg_v)
    pltpu.make_async_copy(h_hbm.at[src_v.at[cur, 0]], r0, s0).wait()
    pltpu.sync_copy(r0, acc.at[dst_v.at[cur, 0]], add=True)

    pltpu.sync_copy(deg_v, deg_hbm.at[wid])

    plsc.subcore_barrier()
    pltpu.sync_copy(
        acc.at[pl.ds(sid * ROWS_PER_TILE, ROWS_PER_TILE)],
        out_hbm.at[cid].at[pl.ds(sid * ROWS_PER_TILE, ROWS_PER_TILE)])


@functools.cache
def _get_sc_agg():
    return functools.partial(
        pl.kernel,
        out_type=(
            jax.ShapeDtypeStruct((NC, NPAD, D), jnp.float32),
            jax.ShapeDtypeStruct((NW, NPAD), jnp.float32),
        ),
        mesh=plsc.VectorSubcoreMesh(core_axis_name="c", subcore_axis_name="s"),
        compiler_params=pltpu.CompilerParams(needs_layout_passes=False),
        scratch_types=[
            pltpu.VMEM_SHARED((NPAD, D), jnp.float32),
            pltpu.VMEM((2, SUB, CH), jnp.int32),
            pltpu.VMEM((2, SUB, CH), jnp.int32),
            pltpu.VMEM((NPAD,), jnp.float32),
            pltpu.VMEM((CH, D), jnp.float32),
            pltpu.VMEM((CH, D), jnp.float32),
            pltpu.SemaphoreType.DMA,
            pltpu.SemaphoreType.DMA,
        ],
    )(_sc_agg_body)


def kernel(x, adj, W1, b1, W2, b2):
    b1r = b1.reshape(1, D)
    b2r = b2.reshape(1, D)
    z = jnp.zeros((CH, D), jnp.float32)

    def _chunked(e):
        e = e.reshape(NW, CHUNKS, CH)
        return jnp.pad(e, ((0, 0), (0, CPAD - CHUNKS), (0, 0)))

    src1 = _chunked(adj[0, 0])
    dst1 = _chunked(adj[0, 1])
    src2 = _chunked(adj[1, 0])
    dst2 = _chunked(adj[1, 1])

    sc_agg = _get_sc_agg()
    h1 = _enc(x, W1, b1r)
    part1, deg1 = sc_agg(h1, src1, dst1, z)
    h2 = _comb_enc(h1, part1, deg1.reshape(NW, NPAD, 1), W2, b2r)
    part2, deg2 = sc_agg(h2, src2, dst2, z)
    return _final(h2, part2, deg2.reshape(NW, NPAD, 1))


# speedup vs baseline: 5.8479x; 1.0817x over previous
import functools

import jax
import jax.numpy as jnp
from jax import lax
from jax.experimental import pallas as pl
from jax.experimental.pallas import tpu as pltpu
from jax.experimental.pallas import tpu_sc as plsc

N = 10000
E = 320000
D = 128
NC = 2
NS = 16
NW = NC * NS
CH = 80
CHUNKS = E // (NW * CH)
SUB = 4
SUPERS = (CHUNKS - 1) // SUB
ROWS_PER_TILE = 10240 // NS
NPAD = NS * ROWS_PER_TILE
BM = 400
GRID = N // BM
L = 16
DBS = 512


def _log0(x):
    n = jnp.sqrt(jnp.sum(x * x, axis=-1, keepdims=True))
    n = jnp.maximum(n, 1e-7)
    return lax.atan2(n, jnp.ones_like(n)) / n * x


def _exp0(u):
    n = jnp.sqrt(jnp.sum(u * u, axis=-1, keepdims=True))
    n = jnp.maximum(n, 1e-7)
    t = jnp.clip(n, 0.0, 1.5)
    return jnp.tan(t) / n * u


def _enc_body(x_ref, w_ref, b_ref, o_ref):
    u = _log0(x_ref[...])
    o_ref[...] = (
        jnp.dot(u, w_ref[...], preferred_element_type=jnp.float32) + b_ref[...]
    )


def _combine(h_ref, p_ref, d_ref):
    h = h_ref[...]
    s = p_ref[0] + p_ref[1]
    d = d_ref[:, pl.ds(0, BM)]
    ones = jnp.ones((NW, 1), jnp.float32)
    dsum = lax.dot_general(d, ones, (((0,), (0,)), ((), ())),
                           preferred_element_type=jnp.float32)
    deg = jnp.maximum(dsum, 1.0)
    o = jax.nn.relu(h + s / deg)
    o = o / (jnp.sqrt(jnp.sum(o * o, axis=-1, keepdims=True)) + 1e-7)
    return _exp0(o)


def _comb_enc_body(h_ref, p_ref, d_ref, w_ref, b_ref, o_ref):
    y = _combine(h_ref, p_ref, d_ref)
    u = _log0(y)
    o_ref[...] = (
        jnp.dot(u, w_ref[...], preferred_element_type=jnp.float32) + b_ref[...]
    )


def _final_body(h_ref, p_ref, d_ref, o_ref):
    o_ref[...] = _combine(h_ref, p_ref, d_ref)


_enc = pl.pallas_call(
    _enc_body,
    grid=(GRID,),
    in_specs=[
        pl.BlockSpec((BM, D), lambda i: (i, 0)),
        pl.BlockSpec((D, D), lambda i: (0, 0)),
        pl.BlockSpec((1, D), lambda i: (0, 0)),
    ],
    out_specs=pl.BlockSpec((BM, D), lambda i: (i, 0)),
    out_shape=jax.ShapeDtypeStruct((N, D), jnp.float32),
    compiler_params=pltpu.CompilerParams(
        dimension_semantics=("parallel",),
    ),
)

_comb_enc = pl.pallas_call(
    _comb_enc_body,
    grid=(GRID,),
    in_specs=[
        pl.BlockSpec((BM, D), lambda i: (i, 0)),
        pl.BlockSpec((NC, BM, D), lambda i: (0, i, 0)),
        pl.BlockSpec((NW, DBS), lambda i: (0, i)),
        pl.BlockSpec((D, D), lambda i: (0, 0)),
        pl.BlockSpec((1, D), lambda i: (0, 0)),
    ],
    out_specs=pl.BlockSpec((BM, D), lambda i: (i, 0)),
    out_shape=jax.ShapeDtypeStruct((N, D), jnp.float32),
    compiler_params=pltpu.CompilerParams(
        dimension_semantics=("parallel",),
    ),
)

_final = pl.pallas_call(
    _final_body,
    grid=(GRID,),
    in_specs=[
        pl.BlockSpec((BM, D), lambda i: (i, 0)),
        pl.BlockSpec((NC, BM, D), lambda i: (0, i, 0)),
        pl.BlockSpec((NW, DBS), lambda i: (0, i)),
    ],
    out_specs=pl.BlockSpec((BM, D), lambda i: (i, 0)),
    out_shape=jax.ShapeDtypeStruct((N, D), jnp.float32),
    compiler_params=pltpu.CompilerParams(
        dimension_semantics=("parallel",),
    ),
)


def _count_deg(dst_v, row, deg_v):
    for k in range(CH // L):
        idx = dst_v[row, pl.ds(k * L, L)]
        pos = (idx // BM) * DBS + lax.rem(idx, BM)
        cnt, last = plsc.scan_count(pos)
        plsc.addupdate_scatter(
            deg_v, [pos], cnt.astype(jnp.float32), mask=last)


def _sc_agg_body(h_hbm, src_hbm, dst_hbm, z_hbm, out_hbm, deg_hbm,
                 acc, src_v, dst_v, deg_v, r0, r1, s0, s1):
    cid = lax.axis_index("c")
    sid = lax.axis_index("s")
    wid = cid * NS + sid
    base = wid * (CHUNKS * CH)

    zero16 = jnp.zeros((L,), jnp.float32)

    def zrow(i, _):
        deg_v[pl.ds(i * L, L)] = zero16
        return 0

    lax.fori_loop(0, (GRID * DBS) // L, zrow, 0)

    pltpu.sync_copy(z_hbm, r0)
    for k in range(ROWS_PER_TILE // CH):
        pltpu.sync_copy(
            r0, acc.at[pl.ds(sid * ROWS_PER_TILE + k * CH, CH)])
    plsc.subcore_barrier()

    def load_idx(row, chunk):
        pltpu.sync_copy(src_hbm.at[pl.ds(base + chunk * CH, CH)],
                        src_v.at[row])
        pltpu.sync_copy(dst_hbm.at[pl.ds(base + chunk * CH, CH)],
                        dst_v.at[row])

    for i in range(SUB):
        load_idx(i, i)
    load_idx(2 * SUB, CHUNKS - 1)
    pltpu.async_copy(h_hbm.at[src_v.at[0]], r0, s0)

    def body(s, _):
        cur = lax.rem(s, 2) * SUB
        nxt = lax.rem(s + 1, 2) * SUB

        @pl.when(s < SUPERS - 1)
        def _():
            for i in range(SUB):
                load_idx(nxt + i, (s + 1) * SUB + i)

        for i in range(SUB):
            rb, sb = (r0, s0) if i % 2 == 0 else (r1, s1)
            nrb, nsb = (r1, s1) if i % 2 == 0 else (r0, s0)
            if i < SUB - 1:
                pltpu.async_copy(h_hbm.at[src_v.at[cur + i + 1]], nrb, nsb)
            else:
                @pl.when(s < SUPERS - 1)
                def _():
                    pltpu.async_copy(h_hbm.at[src_v.at[nxt]], nrb, nsb)

                @pl.when(s == SUPERS - 1)
                def _():
                    pltpu.async_copy(h_hbm.at[src_v.at[2 * SUB]], nrb, nsb)
            _count_deg(dst_v, cur + i, deg_v)
            pltpu.make_async_copy(h_hbm.at[src_v.at[cur + i]], rb, sb).wait()
            pltpu.sync_copy(rb, acc.at[dst_v.at[cur + i]], add=True)
        return 0

    lax.fori_loop(0, SUPERS, body, 0)
    _count_deg(dst_v, 2 * SUB, deg_v)
    pltpu.make_async_copy(h_hbm.at[src_v.at[2 * SUB]], r0, s0).wait()
    pltpu.sync_copy(r0, acc.at[dst_v.at[2 * SUB]], add=True)

    pltpu.sync_copy(deg_v, deg_hbm.at[wid])

    plsc.subcore_barrier()
    pltpu.sync_copy(
        acc.at[pl.ds(sid * ROWS_PER_TILE, ROWS_PER_TILE)],
        out_hbm.at[cid].at[pl.ds(sid * ROWS_PER_TILE, ROWS_PER_TILE)])


@functools.cache
def _get_sc_agg():
    return functools.partial(
        pl.kernel,
        out_type=(
            jax.ShapeDtypeStruct((NC, NPAD, D), jnp.float32),
            jax.ShapeDtypeStruct((NW, GRID * DBS), jnp.float32),
        ),
        mesh=plsc.VectorSubcoreMesh(core_axis_name="c", subcore_axis_name="s"),
        compiler_params=pltpu.CompilerParams(needs_layout_passes=False),
        scratch_types=[
            pltpu.VMEM_SHARED((NPAD, D), jnp.float32),
            pltpu.VMEM((2 * SUB + 1, CH), jnp.int32),
            pltpu.VMEM((2 * SUB + 1, CH), jnp.int32),
            pltpu.VMEM((GRID * DBS,), jnp.float32),
            pltpu.VMEM((CH, D), jnp.float32),
            pltpu.VMEM((CH, D), jnp.float32),
            pltpu.SemaphoreType.DMA,
            pltpu.SemaphoreType.DMA,
        ],
    )(_sc_agg_body)


def kernel(x, adj, W1, b1, W2, b2):
    b1r = b1.reshape(1, D)
    b2r = b2.reshape(1, D)
    z = jnp.zeros((CH, D), jnp.float32)

    sc_agg = _get_sc_agg()
    h1 = _enc(x, W1, b1r)
    part1, deg1 = sc_agg(h1, adj[0, 0], adj[0, 1], z)
    h2 = _comb_enc(h1, part1, deg1, W2, b2r)
    part2, deg2 = sc_agg(h2, adj[1, 0], adj[1, 1], z)
    return _final(h2, part2, deg2)


# speedup vs baseline: 7.8721x; 1.3461x over previous
import functools

import jax
import jax.numpy as jnp
from jax import lax
from jax.experimental import pallas as pl
from jax.experimental.pallas import tpu as pltpu
from jax.experimental.pallas import tpu_sc as plsc

N = 10000
E = 320000
D = 128
NC = 2
NS = 16
NW = NC * NS
CH = 80
CHUNKS = E // (NW * CH)
SUB = 8
SUPERS = CHUNKS // SUB
EPI = CHUNKS - SUPERS * SUB
PADC = 128
ROWS_PER_TILE = 10240 // NS
NPAD = NS * ROWS_PER_TILE
BM = 400
GRID = N // BM
L = 16
DBS = 512


def _log0(x):
    n = jnp.sqrt(jnp.sum(x * x, axis=-1, keepdims=True))
    n = jnp.maximum(n, 1e-7)
    return lax.atan2(n, jnp.ones_like(n)) / n * x


def _exp0(u):
    n = jnp.sqrt(jnp.sum(u * u, axis=-1, keepdims=True))
    n = jnp.maximum(n, 1e-7)
    t = jnp.clip(n, 0.0, 1.5)
    return jnp.tan(t) / n * u


def _enc_body(x_ref, w_ref, b_ref, o_ref):
    u = _log0(x_ref[...])
    o_ref[...] = (
        jnp.dot(u, w_ref[...], preferred_element_type=jnp.float32) + b_ref[...]
    )


def _combine(h_ref, p_ref, d_ref):
    h = h_ref[...]
    s = p_ref[0] + p_ref[1]
    d = d_ref[:, pl.ds(0, BM)]
    ones = jnp.ones((NW, 1), jnp.float32)
    dsum = lax.dot_general(d, ones, (((0,), (0,)), ((), ())),
                           preferred_element_type=jnp.float32)
    deg = jnp.maximum(dsum, 1.0)
    o = jax.nn.relu(h + s / deg)
    o = o / (jnp.sqrt(jnp.sum(o * o, axis=-1, keepdims=True)) + 1e-7)
    return _exp0(o)


def _comb_enc_body(h_ref, p_ref, d_ref, w_ref, b_ref, o_ref):
    y = _combine(h_ref, p_ref, d_ref)
    u = _log0(y)
    o_ref[...] = (
        jnp.dot(u, w_ref[...], preferred_element_type=jnp.float32) + b_ref[...]
    )


def _final_body(h_ref, p_ref, d_ref, o_ref):
    o_ref[...] = _combine(h_ref, p_ref, d_ref)


_enc = pl.pallas_call(
    _enc_body,
    grid=(GRID,),
    in_specs=[
        pl.BlockSpec((BM, D), lambda i: (i, 0)),
        pl.BlockSpec((D, D), lambda i: (0, 0)),
        pl.BlockSpec((1, D), lambda i: (0, 0)),
    ],
    out_specs=pl.BlockSpec((BM, D), lambda i: (i, 0)),
    out_shape=jax.ShapeDtypeStruct((N, D), jnp.float32),
    compiler_params=pltpu.CompilerParams(
        dimension_semantics=("parallel",),
    ),
)

_comb_enc = pl.pallas_call(
    _comb_enc_body,
    grid=(GRID,),
    in_specs=[
        pl.BlockSpec((BM, D), lambda i: (i, 0)),
        pl.BlockSpec((NC, BM, D), lambda i: (0, i, 0)),
        pl.BlockSpec((NW, DBS), lambda i: (0, i)),
        pl.BlockSpec((D, D), lambda i: (0, 0)),
        pl.BlockSpec((1, D), lambda i: (0, 0)),
    ],
    out_specs=pl.BlockSpec((BM, D), lambda i: (i, 0)),
    out_shape=jax.ShapeDtypeStruct((N, D), jnp.float32),
    compiler_params=pltpu.CompilerParams(
        dimension_semantics=("parallel",),
    ),
)

_final = pl.pallas_call(
    _final_body,
    grid=(GRID,),
    in_specs=[
        pl.BlockSpec((BM, D), lambda i: (i, 0)),
        pl.BlockSpec((NC, BM, D), lambda i: (0, i, 0)),
        pl.BlockSpec((NW, DBS), lambda i: (0, i)),
    ],
    out_specs=pl.BlockSpec((BM, D), lambda i: (i, 0)),
    out_shape=jax.ShapeDtypeStruct((N, D), jnp.float32),
    compiler_params=pltpu.CompilerParams(
        dimension_semantics=("parallel",),
    ),
)


def _count_deg(dst_v, row, deg_v):
    for k in range(CH // L):
        idx = dst_v[row, pl.ds(k * L, L)]
        pos = (idx // BM) * DBS + lax.rem(idx, BM)
        cnt, last = plsc.scan_count(pos)
        plsc.addupdate_scatter(
            deg_v, [pos], cnt.astype(jnp.float32), mask=last)


def _sc_agg_body(h_hbm, src_hbm, dst_hbm, z_hbm, out_hbm, deg_hbm,
                 acc, src_v, dst_v, deg_v, r0, r1, s0, s1):
    cid = lax.axis_index("c")
    sid = lax.axis_index("s")
    wid = cid * NS + sid
    base = wid * PADC

    zero16 = jnp.zeros((L,), jnp.float32)

    def zrow(i, _):
        deg_v[pl.ds(i * L, L)] = zero16
        return 0

    lax.fori_loop(0, (GRID * DBS) // L, zrow, 0)

    pltpu.sync_copy(z_hbm, r0)
    for k in range(ROWS_PER_TILE // CH):
        pltpu.sync_copy(
            r0, acc.at[pl.ds(sid * ROWS_PER_TILE + k * CH, CH)])
    plsc.subcore_barrier()

    def load_idx(row, chunk, n):
        pltpu.sync_copy(src_hbm.at[pl.ds(base + chunk, n)],
                        src_v.at[pl.ds(row, n)])
        pltpu.sync_copy(dst_hbm.at[pl.ds(base + chunk, n)],
                        dst_v.at[pl.ds(row, n)])

    load_idx(0, 0, SUB)
    load_idx(2 * SUB, SUPERS * SUB, SUB)
    pltpu.async_copy(h_hbm.at[src_v.at[0]], r0, s0)

    def body(s, _):
        cur = lax.rem(s, 2) * SUB
        nxt = lax.rem(s + 1, 2) * SUB

        @pl.when(s < SUPERS - 1)
        def _():
            load_idx(nxt, (s + 1) * SUB, SUB)

        for i in range(SUB):
            rb, sb = (r0, s0) if i % 2 == 0 else (r1, s1)
            nrb, nsb = (r1, s1) if i % 2 == 0 else (r0, s0)
            if i < SUB - 1:
                pltpu.async_copy(h_hbm.at[src_v.at[cur + i + 1]], nrb, nsb)
            else:
                @pl.when(s < SUPERS - 1)
                def _():
                    pltpu.async_copy(h_hbm.at[src_v.at[nxt]], nrb, nsb)

                @pl.when(s == SUPERS - 1)
                def _():
                    pltpu.async_copy(h_hbm.at[src_v.at[2 * SUB]], nrb, nsb)
            _count_deg(dst_v, cur + i, deg_v)
            pltpu.make_async_copy(h_hbm.at[src_v.at[cur + i]], rb, sb).wait()
            pltpu.sync_copy(rb, acc.at[dst_v.at[cur + i]], add=True)
        return 0

    lax.fori_loop(0, SUPERS, body, 0)
    for e in range(EPI):
        rb, sb = (r0, s0) if e % 2 == 0 else (r1, s1)
        nrb, nsb = (r1, s1) if e % 2 == 0 else (r0, s0)
        if e < EPI - 1:
            pltpu.async_copy(h_hbm.at[src_v.at[2 * SUB + e + 1]], nrb, nsb)
        _count_deg(dst_v, 2 * SUB + e, deg_v)
        pltpu.make_async_copy(h_hbm.at[src_v.at[2 * SUB + e]], rb, sb).wait()
        pltpu.sync_copy(rb, acc.at[dst_v.at[2 * SUB + e]], add=True)

    pltpu.sync_copy(deg_v, deg_hbm.at[wid])

    plsc.subcore_barrier()
    pltpu.sync_copy(
        acc.at[pl.ds(sid * ROWS_PER_TILE, ROWS_PER_TILE)],
        out_hbm.at[cid].at[pl.ds(sid * ROWS_PER_TILE, ROWS_PER_TILE)])


@functools.cache
def _get_sc_agg():
    return functools.partial(
        pl.kernel,
        out_type=(
            jax.ShapeDtypeStruct((NC, NPAD, D), jnp.float32),
            jax.ShapeDtypeStruct((NW, GRID * DBS), jnp.float32),
        ),
        mesh=plsc.VectorSubcoreMesh(core_axis_name="c", subcore_axis_name="s"),
        compiler_params=pltpu.CompilerParams(needs_layout_passes=False),
        scratch_types=[
            pltpu.VMEM_SHARED((NPAD, D), jnp.float32),
            pltpu.VMEM((3 * SUB, CH), jnp.int32),
            pltpu.VMEM((3 * SUB, CH), jnp.int32),
            pltpu.VMEM((GRID * DBS,), jnp.float32),
            pltpu.VMEM((CH, D), jnp.float32),
            pltpu.VMEM((CH, D), jnp.float32),
            pltpu.SemaphoreType.DMA,
            pltpu.SemaphoreType.DMA,
        ],
    )(_sc_agg_body)


def kernel(x, adj, W1, b1, W2, b2):
    b1r = b1.reshape(1, D)
    b2r = b2.reshape(1, D)
    z = jnp.zeros((CH, D), jnp.float32)

    sc_agg = _get_sc_agg()
    h1 = _enc(x, W1, b1r)
    er = adj.reshape(2, 2, NW, CHUNKS, CH)
    er = jnp.pad(er, ((0, 0), (0, 0), (0, 0), (0, PADC - CHUNKS), (0, 0)))
    er = er.reshape(2, 2, NW * PADC, CH)
    e1 = er[0]
    e2 = er[1]
    part1, deg1 = sc_agg(h1, e1[0], e1[1], z)
    h2 = _comb_enc(h1, part1, deg1, W2, b2r)
    part2, deg2 = sc_agg(h2, e2[0], e2[1], z)
    return _final(h2, part2, deg2)
